# baseline (speedup 1.0000x reference)
import functools

import jax
import jax.numpy as jnp
from jax import lax
from jax.experimental import pallas as pl
from jax.experimental.pallas import tpu as pltpu
from jax.experimental.pallas import tpu_sc as plsc

N = 10000
E = 320000
D_IN = 128
D_OUT = 128
K = 3

NC = 2
NS = 16
NW = NC * NS
EPW = E // NW
DCH = 80
CH = 80
NCHUNK = EPW // CH
ROWS_A = 640
ROWS_LAST = N - (NS - 1) * ROWS_A
EV = 80

_f32 = jnp.float32


def _sc_mesh():
    return plsc.VectorSubcoreMesh(
        core_axis_name="c", subcore_axis_name="s",
        num_cores=NC, num_subcores=NS)


def _sc_degrees(eidx_r):
    nch_e = E // NS // DCH

    @functools.partial(
        pl.kernel,
        out_type=(jax.ShapeDtypeStruct((N,), _f32),
                  jax.ShapeDtypeStruct((N,), _f32)),
        mesh=_sc_mesh(),
        scratch_types=[
            pltpu.VMEM((nch_e, DCH), jnp.int32),
            pltpu.VMEM((DCH,), _f32),
            pltpu.VMEM((EV,), _f32),
            pltpu.VMEM_SHARED((N,), _f32),
            pltpu.SemaphoreType.DMA,
        ],
    )
    def deg_kernel(eidx, out_deg, in_deg, idx_v, ones_v, buf_v, deg_sp, sem):
        c = lax.axis_index("c")
        t = lax.axis_index("s")
        base = t * ROWS_A
        nzero = jnp.where(t < NS - 1, ROWS_A // EV, ROWS_LAST // EV)

        for i in range(DCH // 16):
            ones_v[pl.ds(i * 16, 16)] = jnp.full((16,), 1.0, _f32)
        for i in range(EV // 16):
            buf_v[pl.ds(i * 16, 16)] = jnp.zeros((16,), _f32)

        def zero_body(i, carry):
            pltpu.sync_copy(buf_v, deg_sp.at[pl.ds(base + i * EV, EV)])
            return carry
        lax.fori_loop(0, nzero, zero_body, 0)

        pltpu.sync_copy(eidx.at[c, t], idx_v)
        plsc.subcore_barrier()

        def edge_body(j, carry):
            pltpu.async_copy(ones_v, deg_sp.at[idx_v.at[j]], sem, add=True)
            return carry
        lax.fori_loop(0, nch_e, edge_body, 0)

        def edge_drain(j, carry):
            pltpu.make_async_copy(ones_v, deg_sp.at[idx_v.at[0]], sem).wait()
            return carry
        lax.fori_loop(0, nch_e, edge_drain, 0)
        plsc.subcore_barrier()

        def evac_body(i, carry):
            off = base + i * EV
            pltpu.sync_copy(deg_sp.at[pl.ds(off, EV)], buf_v)

            @pl.when(c == 0)
            def _():
                pltpu.sync_copy(buf_v, out_deg.at[pl.ds(off, EV)])

            @pl.when(c == 1)
            def _():
                pltpu.sync_copy(buf_v, in_deg.at[pl.ds(off, EV)])
            return carry
        lax.fori_loop(0, nzero, evac_body, 0)

    return deg_kernel(eidx_r)


def _sc_hop(z, src_r, dst_r):
    @functools.partial(
        pl.kernel,
        out_type=(jax.ShapeDtypeStruct((N, D_OUT), _f32),
                  jax.ShapeDtypeStruct((N, D_OUT), _f32)),
        mesh=_sc_mesh(),
        scratch_types=[
            pltpu.VMEM((EPW,), jnp.int32),
            pltpu.VMEM((NCHUNK, CH), jnp.int32),
            pltpu.VMEM((CH, D_OUT), _f32),
            pltpu.VMEM((CH, D_OUT), _f32),
            pltpu.VMEM_SHARED((N, D_OUT), _f32),
            pltpu.SemaphoreType.DMA,
            pltpu.SemaphoreType.DMA,
            pltpu.SemaphoreType.DMA,
            pltpu.SemaphoreType.DMA,
        ],
    )
    def hop_kernel(zr, srcr, dstr, p0, p1,
                   src_v, dst_v, rows_a, rows_b, acc_sp,
                   sem_a, sem_b, sem_sa, sem_sb):
        c = lax.axis_index("c")
        t = lax.axis_index("s")
        w = c * NS + t
        base = t * ROWS_A

        for i in range(CH):
            for q in range(D_OUT // 16):
                rows_a[i, pl.ds(q * 16, 16)] = jnp.zeros((16,), _f32)

        pltpu.async_copy(srcr.at[w], src_v, sem_b)
        pltpu.async_copy(dstr.at[w], dst_v, sem_b)
        nz = jnp.where(t < NS - 1, ROWS_A // CH, ROWS_LAST // CH)

        def zero_issue(i, carry):
            pltpu.async_copy(rows_a, acc_sp.at[pl.ds(base + i * CH, CH), :],
                             sem_a)
            return carry
        lax.fori_loop(0, nz, zero_issue, 0)

        def zero_drain(i, carry):
            pltpu.make_async_copy(rows_a, acc_sp.at[pl.ds(base, CH), :],
                                  sem_a).wait()
            return carry
        lax.fori_loop(0, nz, zero_drain, 0)
        pltpu.make_async_copy(srcr.at[w], src_v, sem_b).wait()
        pltpu.make_async_copy(dstr.at[w], dst_v, sem_b).wait()
        plsc.subcore_barrier()

        def src_ix(j):
            return src_v.at[pl.ds(j * CH, CH)]

        pltpu.async_copy(zr.at[src_ix(0)], rows_a, sem_a)
        pltpu.async_copy(zr.at[src_ix(1)], rows_b, sem_b)

        def edge_body(i, carry):
            j = i * 2
            pltpu.make_async_copy(zr.at[src_ix(j)], rows_a, sem_a).wait()
            pltpu.async_copy(rows_a, acc_sp.at[dst_v.at[j]], sem_sa, add=True)
            pltpu.make_async_copy(zr.at[src_ix(j + 1)], rows_b, sem_b).wait()
            pltpu.async_copy(rows_b, acc_sp.at[dst_v.at[j + 1]], sem_sb,
                             add=True)
            pltpu.make_async_copy(rows_a, acc_sp.at[dst_v.at[j]],
                                  sem_sa).wait()
            pltpu.async_copy(zr.at[src_ix(j + 2)], rows_a, sem_a)
            pltpu.make_async_copy(rows_b, acc_sp.at[dst_v.at[j + 1]],
                                  sem_sb).wait()

            @pl.when(j + 3 < NCHUNK)
            def _():
                pltpu.async_copy(zr.at[src_ix(j + 3)], rows_b, sem_b)
            return carry
        lax.fori_loop(0, (NCHUNK - 1) // 2, edge_body, 0)
        pltpu.make_async_copy(zr.at[src_ix(NCHUNK - 1)], rows_a, sem_a).wait()
        pltpu.sync_copy(rows_a, acc_sp.at[dst_v.at[NCHUNK - 1]], add=True)
        plsc.subcore_barrier()

        nev = jnp.where(t < NS - 1, ROWS_A // CH, ROWS_LAST // CH)

        def evac_body(i, carry):
            j = i * 2
            off = base + j * CH
            pltpu.sync_copy(acc_sp.at[pl.ds(off, CH), :], rows_a)

            @pl.when(c == 0)
            def _():
                pltpu.async_copy(rows_a, p0.at[pl.ds(off, CH), :], sem_a)

            @pl.when(c == 1)
            def _():
                pltpu.async_copy(rows_a, p1.at[pl.ds(off, CH), :], sem_a)

            @pl.when(j + 1 < nev)
            def _():
                off2 = base + (j + 1) * CH
                pltpu.sync_copy(acc_sp.at[pl.ds(off2, CH), :], rows_b)

                @pl.when(c == 0)
                def _():
                    pltpu.async_copy(rows_b, p0.at[pl.ds(off2, CH), :], sem_b)

                @pl.when(c == 1)
                def _():
                    pltpu.async_copy(rows_b, p1.at[pl.ds(off2, CH), :], sem_b)
            pltpu.make_async_copy(rows_a, p0.at[pl.ds(base, CH), :],
                                  sem_a).wait()

            @pl.when(j + 1 < nev)
            def _():
                pltpu.make_async_copy(rows_b, p0.at[pl.ds(base, CH), :],
                                      sem_b).wait()
            return carry
        lax.fori_loop(0, (nev + 1) // 2, evac_body, 0)

    return hop_kernel(z, src_r, dst_r)


BN = 1000


def _mm_body(x_ref, w_ref, od_ref, h_ref, z_ref):
    h = jnp.dot(x_ref[...].astype(jnp.bfloat16),
                w_ref[...].astype(jnp.bfloat16),
                preferred_element_type=_f32)
    h_ref[...] = h
    inv_out = 1.0 / jnp.sqrt(jnp.maximum(od_ref[...], 1.0))
    z_ref[...] = h * inv_out


def _tc_matmul(n_feat, weight, out_deg2):
    return pl.pallas_call(
        _mm_body,
        grid=(N // BN,),
        in_specs=[
            pl.BlockSpec((BN, D_IN), lambda i: (i, 0)),
            pl.BlockSpec((D_IN, D_OUT), lambda i: (0, 0)),
            pl.BlockSpec((BN, 1), lambda i: (i, 0)),
        ],
        out_specs=(
            pl.BlockSpec((BN, D_OUT), lambda i: (i, 0)),
            pl.BlockSpec((BN, D_OUT), lambda i: (i, 0)),
        ),
        out_shape=(
            jax.ShapeDtypeStruct((N, D_OUT), _f32),
            jax.ShapeDtypeStruct((N, D_OUT), _f32),
        ),
    )(n_feat, weight, out_deg2)


def _scale_body(p0_ref, p1_ref, id_ref, od_ref, ak_ref, z_ref):
    agg = p0_ref[...] + p1_ref[...]
    inv_in = 1.0 / jnp.sqrt(jnp.maximum(id_ref[...], 1.0))
    inv_out = 1.0 / jnp.sqrt(jnp.maximum(od_ref[...], 1.0))
    ak = agg * inv_in
    ak_ref[...] = ak
    z_ref[...] = ak * inv_out


def _tc_scale(p0, p1, in_deg2, out_deg2):
    return pl.pallas_call(
        _scale_body,
        grid=(N // BN,),
        in_specs=[
            pl.BlockSpec((BN, D_OUT), lambda i: (i, 0)),
            pl.BlockSpec((BN, D_OUT), lambda i: (i, 0)),
            pl.BlockSpec((BN, 1), lambda i: (i, 0)),
            pl.BlockSpec((BN, 1), lambda i: (i, 0)),
        ],
        out_specs=(
            pl.BlockSpec((BN, D_OUT), lambda i: (i, 0)),
            pl.BlockSpec((BN, D_OUT), lambda i: (i, 0)),
        ),
        out_shape=(
            jax.ShapeDtypeStruct((N, D_OUT), _f32),
            jax.ShapeDtypeStruct((N, D_OUT), _f32),
        ),
    )(p0, p1, in_deg2, out_deg2)


def _se_body(a0_ref, a1_ref, a2_ref, p0_ref, p1_ref, id_ref,
             att_ref, w1_ref, w2_ref, out_ref):
    inv_in = 1.0 / jnp.sqrt(jnp.maximum(id_ref[...], 1.0))
    a3 = (p0_ref[...] + p1_ref[...]) * inv_in
    blks = [a0_ref[...], a1_ref[...], a2_ref[...], a3]
    att = att_ref[...].reshape(1, D_OUT)
    s = jnp.concatenate(
        [jnp.sum(b * att, axis=1, keepdims=True) for b in blks], axis=1)
    norm = jnp.sqrt(jnp.sum(s * s, axis=1, keepdims=True))
    s = s / jnp.maximum(norm, 1e-12)
    e = jnp.clip(jnp.dot(s.astype(jnp.bfloat16),
                         w1_ref[...].astype(jnp.bfloat16),
                         preferred_element_type=_f32), 0.0, 6.0)
    e = jnp.tanh(jnp.dot(e.astype(jnp.bfloat16),
                         w2_ref[...].astype(jnp.bfloat16),
                         preferred_element_type=_f32))
    out = blks[0] * e[:, 0:1]
    for k in range(1, K + 1):
        out = out + blks[k] * e[:, k:k + 1]
    out_ref[...] = out


def _tc_se(a0, a1, a2, p0, p1, in_deg2, e_att, e_w1, e_w2):
    return pl.pallas_call(
        _se_body,
        grid=(N // BN,),
        in_specs=[
            pl.BlockSpec((BN, D_OUT), lambda i: (i, 0)),
            pl.BlockSpec((BN, D_OUT), lambda i: (i, 0)),
            pl.BlockSpec((BN, D_OUT), lambda i: (i, 0)),
            pl.BlockSpec((BN, D_OUT), lambda i: (i, 0)),
            pl.BlockSpec((BN, D_OUT), lambda i: (i, 0)),
            pl.BlockSpec((BN, 1), lambda i: (i, 0)),
            pl.BlockSpec((D_OUT, 1), lambda i: (0, 0)),
            pl.BlockSpec((K + 1, K + 1), lambda i: (0, 0)),
            pl.BlockSpec((K + 1, K + 1), lambda i: (0, 0)),
        ],
        out_specs=pl.BlockSpec((BN, D_OUT), lambda i: (i, 0)),
        out_shape=jax.ShapeDtypeStruct((N, D_OUT), _f32),
    )(a0, a1, a2, p0, p1, in_deg2, e_att, e_w1, e_w2)


def kernel(n_feat, edge_index, weight, e_weight1, e_weight2, e_att):
    deg_idx = edge_index.reshape(2, NS, E // NS // DCH, DCH)
    out_deg, in_deg = _sc_degrees(deg_idx)
    od2 = out_deg.reshape(N, 1)
    id2 = in_deg.reshape(N, 1)

    h, z = _tc_matmul(n_feat, weight, od2)

    src_r = edge_index[0].reshape(NW, EPW)
    dst_r = edge_index[1].reshape(NW, NCHUNK, CH)
    aggr = [h]
    for _ in range(K - 1):
        p0, p1 = _sc_hop(z, src_r, dst_r)
        ak, z = _tc_scale(p0, p1, id2, od2)
        aggr.append(ak)
    p0, p1 = _sc_hop(z, src_r, dst_r)

    return _tc_se(aggr[0], aggr[1], aggr[2], p0, p1, id2,
                  e_att, e_weight1, e_weight2)


# speedup vs baseline: 1.2337x; 1.2337x over previous
import functools

import jax
import jax.numpy as jnp
from jax import lax
from jax.experimental import pallas as pl
from jax.experimental.pallas import tpu as pltpu
from jax.experimental.pallas import tpu_sc as plsc

N = 10000
E = 320000
D_IN = 128
D_OUT = 128
K = 3

NC = 2
NS = 16
NW = NC * NS
EPW = E // NW
DCH = 80
CH = 80
NCHUNK = EPW // CH
ROWS_A = 640
ROWS_LAST = N - (NS - 1) * ROWS_A
EV = 80

_f32 = jnp.float32


def _sc_mesh():
    return plsc.VectorSubcoreMesh(
        core_axis_name="c", subcore_axis_name="s",
        num_cores=NC, num_subcores=NS)


def _sc_degrees(eidx_r):
    nch_e = E // NS // DCH

    @functools.partial(
        pl.kernel,
        out_type=(jax.ShapeDtypeStruct((N,), _f32),
                  jax.ShapeDtypeStruct((N,), _f32)),
        mesh=_sc_mesh(),
        scratch_types=[
            pltpu.VMEM((nch_e, DCH), jnp.int32),
            pltpu.VMEM((DCH,), _f32),
            pltpu.VMEM((EV,), _f32),
            pltpu.VMEM_SHARED((N,), _f32),
            pltpu.SemaphoreType.DMA,
        ],
    )
    def deg_kernel(eidx, out_deg, in_deg, idx_v, ones_v, buf_v, deg_sp, sem):
        c = lax.axis_index("c")
        t = lax.axis_index("s")
        base = t * ROWS_A
        nzero = jnp.where(t < NS - 1, ROWS_A // EV, ROWS_LAST // EV)

        for i in range(DCH // 16):
            ones_v[pl.ds(i * 16, 16)] = jnp.full((16,), 1.0, _f32)
        for i in range(EV // 16):
            buf_v[pl.ds(i * 16, 16)] = jnp.zeros((16,), _f32)

        def zero_body(i, carry):
            pltpu.sync_copy(buf_v, deg_sp.at[pl.ds(base + i * EV, EV)])
            return carry
        lax.fori_loop(0, nzero, zero_body, 0)

        pltpu.sync_copy(eidx.at[c, t], idx_v)
        plsc.subcore_barrier()

        def edge_body(j, carry):
            pltpu.async_copy(ones_v, deg_sp.at[idx_v.at[j]], sem, add=True)
            return carry
        lax.fori_loop(0, nch_e, edge_body, 0)

        def edge_drain(j, carry):
            pltpu.make_async_copy(ones_v, deg_sp.at[idx_v.at[0]], sem).wait()
            return carry
        lax.fori_loop(0, nch_e, edge_drain, 0)
        plsc.subcore_barrier()

        def evac_body(i, carry):
            off = base + i * EV
            pltpu.sync_copy(deg_sp.at[pl.ds(off, EV)], buf_v)

            @pl.when(c == 0)
            def _():
                pltpu.sync_copy(buf_v, out_deg.at[pl.ds(off, EV)])

            @pl.when(c == 1)
            def _():
                pltpu.sync_copy(buf_v, in_deg.at[pl.ds(off, EV)])
            return carry
        lax.fori_loop(0, nzero, evac_body, 0)

    return deg_kernel(eidx_r)


def _sc_hop(z, src_r, dst_r):
    @functools.partial(
        pl.kernel,
        out_type=(jax.ShapeDtypeStruct((N, D_OUT), _f32),
                  jax.ShapeDtypeStruct((N, D_OUT), _f32)),
        mesh=_sc_mesh(),
        scratch_types=[
            pltpu.VMEM((EPW,), jnp.int32),
            pltpu.VMEM((NCHUNK, CH), jnp.int32),
            pltpu.VMEM((CH, D_OUT), _f32),
            pltpu.VMEM((CH, D_OUT), _f32),
            pltpu.VMEM_SHARED((N, D_OUT), _f32),
            pltpu.SemaphoreType.DMA,
            pltpu.SemaphoreType.DMA,
            pltpu.SemaphoreType.DMA,
            pltpu.SemaphoreType.DMA,
        ],
    )
    def hop_kernel(zr, srcr, dstr, p0, p1,
                   src_v, dst_v, rows_a, rows_b, acc_sp,
                   sem_a, sem_b, sem_sa, sem_sb):
        c = lax.axis_index("c")
        t = lax.axis_index("s")
        w = c * NS + t
        base = t * ROWS_A

        for i in range(CH):
            for q in range(D_OUT // 16):
                rows_a[i, pl.ds(q * 16, 16)] = jnp.zeros((16,), _f32)

        pltpu.async_copy(srcr.at[w], src_v, sem_b)
        pltpu.async_copy(dstr.at[w], dst_v, sem_b)
        nz = jnp.where(t < NS - 1, ROWS_A // CH, ROWS_LAST // CH)

        def zero_issue(i, carry):
            pltpu.async_copy(rows_a, acc_sp.at[pl.ds(base + i * CH, CH), :],
                             sem_a)
            return carry
        lax.fori_loop(0, nz, zero_issue, 0)

        def zero_drain(i, carry):
            pltpu.make_async_copy(rows_a, acc_sp.at[pl.ds(base, CH), :],
                                  sem_a).wait()
            return carry
        lax.fori_loop(0, nz, zero_drain, 0)
        pltpu.make_async_copy(srcr.at[w], src_v, sem_b).wait()
        pltpu.make_async_copy(dstr.at[w], dst_v, sem_b).wait()
        plsc.subcore_barrier()

        def src_ix(j):
            return src_v.at[pl.ds(j * CH, CH)]

        pltpu.async_copy(zr.at[src_ix(0)], rows_a, sem_a)

        def edge_body(i, carry):
            j = i * 2
            pltpu.async_copy(zr.at[src_ix(j + 1)], rows_b, sem_b)
            pltpu.make_async_copy(zr.at[src_ix(j)], rows_a, sem_a).wait()
            pltpu.sync_copy(rows_a, acc_sp.at[dst_v.at[j]], add=True)
            pltpu.async_copy(zr.at[src_ix(j + 2)], rows_a, sem_a)
            pltpu.make_async_copy(zr.at[src_ix(j + 1)], rows_b, sem_b).wait()
            pltpu.sync_copy(rows_b, acc_sp.at[dst_v.at[j + 1]], add=True)
            return carry
        lax.fori_loop(0, (NCHUNK - 1) // 2, edge_body, 0)
        pltpu.make_async_copy(zr.at[src_ix(NCHUNK - 1)], rows_a, sem_a).wait()
        pltpu.sync_copy(rows_a, acc_sp.at[dst_v.at[NCHUNK - 1]], add=True)
        plsc.subcore_barrier()

        nev = jnp.where(t < NS - 1, ROWS_A // CH, ROWS_LAST // CH)

        def evac_body(i, carry):
            j = i * 2
            off = base + j * CH
            pltpu.sync_copy(acc_sp.at[pl.ds(off, CH), :], rows_a)

            @pl.when(c == 0)
            def _():
                pltpu.async_copy(rows_a, p0.at[pl.ds(off, CH), :], sem_a)

            @pl.when(c == 1)
            def _():
                pltpu.async_copy(rows_a, p1.at[pl.ds(off, CH), :], sem_a)

            @pl.when(j + 1 < nev)
            def _():
                off2 = base + (j + 1) * CH
                pltpu.sync_copy(acc_sp.at[pl.ds(off2, CH), :], rows_b)

                @pl.when(c == 0)
                def _():
                    pltpu.async_copy(rows_b, p0.at[pl.ds(off2, CH), :], sem_b)

                @pl.when(c == 1)
                def _():
                    pltpu.async_copy(rows_b, p1.at[pl.ds(off2, CH), :], sem_b)
            pltpu.make_async_copy(rows_a, p0.at[pl.ds(base, CH), :],
                                  sem_a).wait()

            @pl.when(j + 1 < nev)
            def _():
                pltpu.make_async_copy(rows_b, p0.at[pl.ds(base, CH), :],
                                      sem_b).wait()
            return carry
        lax.fori_loop(0, (nev + 1) // 2, evac_body, 0)

    return hop_kernel(z, src_r, dst_r)


BN = 1000


def _mm_body(x_ref, w_ref, od_ref, h_ref, z_ref):
    h = jnp.dot(x_ref[...].astype(jnp.bfloat16),
                w_ref[...].astype(jnp.bfloat16),
                preferred_element_type=_f32)
    h_ref[...] = h
    inv_out = 1.0 / jnp.sqrt(jnp.maximum(od_ref[...], 1.0))
    z_ref[...] = h * inv_out


def _tc_matmul(n_feat, weight, out_deg2):
    return pl.pallas_call(
        _mm_body,
        grid=(N // BN,),
        in_specs=[
            pl.BlockSpec((BN, D_IN), lambda i: (i, 0)),
            pl.BlockSpec((D_IN, D_OUT), lambda i: (0, 0)),
            pl.BlockSpec((BN, 1), lambda i: (i, 0)),
        ],
        out_specs=(
            pl.BlockSpec((BN, D_OUT), lambda i: (i, 0)),
            pl.BlockSpec((BN, D_OUT), lambda i: (i, 0)),
        ),
        out_shape=(
            jax.ShapeDtypeStruct((N, D_OUT), _f32),
            jax.ShapeDtypeStruct((N, D_OUT), _f32),
        ),
    )(n_feat, weight, out_deg2)


def _scale_body(p0_ref, p1_ref, id_ref, od_ref, ak_ref, z_ref):
    agg = p0_ref[...] + p1_ref[...]
    inv_in = 1.0 / jnp.sqrt(jnp.maximum(id_ref[...], 1.0))
    inv_out = 1.0 / jnp.sqrt(jnp.maximum(od_ref[...], 1.0))
    ak = agg * inv_in
    ak_ref[...] = ak
    z_ref[...] = ak * inv_out


def _tc_scale(p0, p1, in_deg2, out_deg2):
    return pl.pallas_call(
        _scale_body,
        grid=(N // BN,),
        in_specs=[
            pl.BlockSpec((BN, D_OUT), lambda i: (i, 0)),
            pl.BlockSpec((BN, D_OUT), lambda i: (i, 0)),
            pl.BlockSpec((BN, 1), lambda i: (i, 0)),
            pl.BlockSpec((BN, 1), lambda i: (i, 0)),
        ],
        out_specs=(
            pl.BlockSpec((BN, D_OUT), lambda i: (i, 0)),
            pl.BlockSpec((BN, D_OUT), lambda i: (i, 0)),
        ),
        out_shape=(
            jax.ShapeDtypeStruct((N, D_OUT), _f32),
            jax.ShapeDtypeStruct((N, D_OUT), _f32),
        ),
    )(p0, p1, in_deg2, out_deg2)


def _se_body(a0_ref, a1_ref, a2_ref, p0_ref, p1_ref, id_ref,
             att_ref, w1_ref, w2_ref, out_ref):
    inv_in = 1.0 / jnp.sqrt(jnp.maximum(id_ref[...], 1.0))
    a3 = (p0_ref[...] + p1_ref[...]) * inv_in
    blks = [a0_ref[...], a1_ref[...], a2_ref[...], a3]
    att = att_ref[...].reshape(1, D_OUT)
    s = jnp.concatenate(
        [jnp.sum(b * att, axis=1, keepdims=True) for b in blks], axis=1)
    norm = jnp.sqrt(jnp.sum(s * s, axis=1, keepdims=True))
    s = s / jnp.maximum(norm, 1e-12)
    e = jnp.clip(jnp.dot(s.astype(jnp.bfloat16),
                         w1_ref[...].astype(jnp.bfloat16),
                         preferred_element_type=_f32), 0.0, 6.0)
    e = jnp.tanh(jnp.dot(e.astype(jnp.bfloat16),
                         w2_ref[...].astype(jnp.bfloat16),
                         preferred_element_type=_f32))
    out = blks[0] * e[:, 0:1]
    for k in range(1, K + 1):
        out = out + blks[k] * e[:, k:k + 1]
    out_ref[...] = out


def _tc_se(a0, a1, a2, p0, p1, in_deg2, e_att, e_w1, e_w2):
    return pl.pallas_call(
        _se_body,
        grid=(N // BN,),
        in_specs=[
            pl.BlockSpec((BN, D_OUT), lambda i: (i, 0)),
            pl.BlockSpec((BN, D_OUT), lambda i: (i, 0)),
            pl.BlockSpec((BN, D_OUT), lambda i: (i, 0)),
            pl.BlockSpec((BN, D_OUT), lambda i: (i, 0)),
            pl.BlockSpec((BN, D_OUT), lambda i: (i, 0)),
            pl.BlockSpec((BN, 1), lambda i: (i, 0)),
            pl.BlockSpec((D_OUT, 1), lambda i: (0, 0)),
            pl.BlockSpec((K + 1, K + 1), lambda i: (0, 0)),
            pl.BlockSpec((K + 1, K + 1), lambda i: (0, 0)),
        ],
        out_specs=pl.BlockSpec((BN, D_OUT), lambda i: (i, 0)),
        out_shape=jax.ShapeDtypeStruct((N, D_OUT), _f32),
    )(a0, a1, a2, p0, p1, in_deg2, e_att, e_w1, e_w2)


def kernel(n_feat, edge_index, weight, e_weight1, e_weight2, e_att):
    deg_idx = edge_index.reshape(2, NS, E // NS // DCH, DCH)
    out_deg, in_deg = _sc_degrees(deg_idx)
    od2 = out_deg.reshape(N, 1)
    id2 = in_deg.reshape(N, 1)

    h, z = _tc_matmul(n_feat, weight, od2)

    src_r = edge_index[0].reshape(NW, EPW)
    dst_r = edge_index[1].reshape(NW, NCHUNK, CH)
    aggr = [h]
    for _ in range(K - 1):
        p0, p1 = _sc_hop(z, src_r, dst_r)
        ak, z = _tc_scale(p0, p1, id2, od2)
        aggr.append(ak)
    p0, p1 = _sc_hop(z, src_r, dst_r)

    return _tc_se(aggr[0], aggr[1], aggr[2], p0, p1, id2,
                  e_att, e_weight1, e_weight2)


# speedup vs baseline: 1.2522x; 1.0150x over previous
import functools

import jax
import jax.numpy as jnp
from jax import lax
from jax.experimental import pallas as pl
from jax.experimental.pallas import tpu as pltpu
from jax.experimental.pallas import tpu_sc as plsc

N = 10000
E = 320000
D_IN = 128
D_OUT = 128
K = 3

NC = 2
NS = 16
NW = NC * NS
EPW = E // NW
DCH = 80
CH = 80
NCHUNK = EPW // CH
ROWS_A = 640
ROWS_LAST = N - (NS - 1) * ROWS_A
EV = 80

_f32 = jnp.float32


def _sc_mesh():
    return plsc.VectorSubcoreMesh(
        core_axis_name="c", subcore_axis_name="s",
        num_cores=NC, num_subcores=NS)


def _sc_degrees(eidx_r):
    nch_e = E // NS // DCH

    @functools.partial(
        pl.kernel,
        out_type=(jax.ShapeDtypeStruct((N,), _f32),
                  jax.ShapeDtypeStruct((N,), _f32)),
        mesh=_sc_mesh(),
        scratch_types=[
            pltpu.VMEM((nch_e, DCH), jnp.int32),
            pltpu.VMEM((DCH,), _f32),
            pltpu.VMEM((EV,), _f32),
            pltpu.VMEM_SHARED((N,), _f32),
            pltpu.SemaphoreType.DMA,
        ],
    )
    def deg_kernel(eidx, out_deg, in_deg, idx_v, ones_v, buf_v, deg_sp, sem):
        c = lax.axis_index("c")
        t = lax.axis_index("s")
        base = t * ROWS_A
        nzero = jnp.where(t < NS - 1, ROWS_A // EV, ROWS_LAST // EV)

        for i in range(DCH // 16):
            ones_v[pl.ds(i * 16, 16)] = jnp.full((16,), 1.0, _f32)
        for i in range(EV // 16):
            buf_v[pl.ds(i * 16, 16)] = jnp.zeros((16,), _f32)

        def zero_body(i, carry):
            pltpu.sync_copy(buf_v, deg_sp.at[pl.ds(base + i * EV, EV)])
            return carry
        lax.fori_loop(0, nzero, zero_body, 0)

        pltpu.sync_copy(eidx.at[c, t], idx_v)
        plsc.subcore_barrier()

        def edge_body(j, carry):
            pltpu.async_copy(ones_v, deg_sp.at[idx_v.at[j]], sem, add=True)
            return carry
        lax.fori_loop(0, nch_e, edge_body, 0)

        def edge_drain(j, carry):
            pltpu.make_async_copy(ones_v, deg_sp.at[idx_v.at[0]], sem).wait()
            return carry
        lax.fori_loop(0, nch_e, edge_drain, 0)
        plsc.subcore_barrier()

        def evac_body(i, carry):
            off = base + i * EV
            pltpu.sync_copy(deg_sp.at[pl.ds(off, EV)], buf_v)

            @pl.when(c == 0)
            def _():
                pltpu.sync_copy(buf_v, out_deg.at[pl.ds(off, EV)])

            @pl.when(c == 1)
            def _():
                pltpu.sync_copy(buf_v, in_deg.at[pl.ds(off, EV)])
            return carry
        lax.fori_loop(0, nzero, evac_body, 0)

    return deg_kernel(eidx_r)


def _sc_hop(z, src_r, dst_r):
    @functools.partial(
        pl.kernel,
        out_type=(jax.ShapeDtypeStruct((N, D_OUT), _f32),
                  jax.ShapeDtypeStruct((N, D_OUT), _f32)),
        mesh=_sc_mesh(),
        scratch_types=[
            pltpu.VMEM((EPW,), jnp.int32),
            pltpu.VMEM((NCHUNK, CH), jnp.int32),
            pltpu.VMEM((CH, D_OUT), _f32),
            pltpu.VMEM((CH, D_OUT), _f32),
            pltpu.VMEM_SHARED((N, D_OUT), _f32),
            pltpu.SemaphoreType.DMA,
            pltpu.SemaphoreType.DMA,
            pltpu.SemaphoreType.DMA,
            pltpu.SemaphoreType.DMA,
        ],
    )
    def hop_kernel(zr, srcr, dstr, p0, p1,
                   src_v, dst_v, rows_a, rows_b, acc_sp,
                   sem_a, sem_b, sem_sa, sem_sb):
        c = lax.axis_index("c")
        t = lax.axis_index("s")
        w = c * NS + t
        base = t * ROWS_A

        for i in range(CH):
            for q in range(D_OUT // 16):
                rows_a[i, pl.ds(q * 16, 16)] = jnp.zeros((16,), _f32)

        pltpu.async_copy(srcr.at[w], src_v, sem_b)
        pltpu.async_copy(dstr.at[w], dst_v, sem_b)
        nz = jnp.where(t < NS - 1, ROWS_A // CH, ROWS_LAST // CH)

        def zero_issue(i, carry):
            pltpu.async_copy(rows_a, acc_sp.at[pl.ds(base + i * CH, CH), :],
                             sem_a)
            return carry
        lax.fori_loop(0, nz, zero_issue, 0)

        def zero_drain(i, carry):
            pltpu.make_async_copy(rows_a, acc_sp.at[pl.ds(base, CH), :],
                                  sem_a).wait()
            return carry
        lax.fori_loop(0, nz, zero_drain, 0)
        pltpu.make_async_copy(srcr.at[w], src_v, sem_b).wait()
        pltpu.make_async_copy(dstr.at[w], dst_v, sem_b).wait()
        plsc.subcore_barrier()

        def src_ix(j):
            return src_v.at[pl.ds(j * CH, CH)]

        pltpu.async_copy(zr.at[src_ix(0)], rows_a, sem_a)

        def edge_body(i, carry):
            j = i * 2
            pltpu.async_copy(zr.at[src_ix(j + 1)], rows_b, sem_b)
            pltpu.make_async_copy(zr.at[src_ix(j)], rows_a, sem_a).wait()
            pltpu.sync_copy(rows_a, acc_sp.at[dst_v.at[j]], add=True)
            pltpu.async_copy(zr.at[src_ix(j + 2)], rows_a, sem_a)
            pltpu.make_async_copy(zr.at[src_ix(j + 1)], rows_b, sem_b).wait()
            pltpu.sync_copy(rows_b, acc_sp.at[dst_v.at[j + 1]], add=True)
            return carry
        lax.fori_loop(0, (NCHUNK - 1) // 2, edge_body, 0)
        pltpu.make_async_copy(zr.at[src_ix(NCHUNK - 1)], rows_a, sem_a).wait()
        pltpu.sync_copy(rows_a, acc_sp.at[dst_v.at[NCHUNK - 1]], add=True)
        plsc.subcore_barrier()

        nev = jnp.where(t < NS - 1, ROWS_A // CH, ROWS_LAST // CH)

        def evac_body(i, carry):
            j = i * 2
            off = base + j * CH
            pltpu.sync_copy(acc_sp.at[pl.ds(off, CH), :], rows_a)

            @pl.when(c == 0)
            def _():
                pltpu.async_copy(rows_a, p0.at[pl.ds(off, CH), :], sem_a)

            @pl.when(c == 1)
            def _():
                pltpu.async_copy(rows_a, p1.at[pl.ds(off, CH), :], sem_a)

            @pl.when(j + 1 < nev)
            def _():
                off2 = base + (j + 1) * CH
                pltpu.sync_copy(acc_sp.at[pl.ds(off2, CH), :], rows_b)

                @pl.when(c == 0)
                def _():
                    pltpu.async_copy(rows_b, p0.at[pl.ds(off2, CH), :], sem_b)

                @pl.when(c == 1)
                def _():
                    pltpu.async_copy(rows_b, p1.at[pl.ds(off2, CH), :], sem_b)
            pltpu.make_async_copy(rows_a, p0.at[pl.ds(base, CH), :],
                                  sem_a).wait()

            @pl.when(j + 1 < nev)
            def _():
                pltpu.make_async_copy(rows_b, p0.at[pl.ds(base, CH), :],
                                      sem_b).wait()
            return carry
        lax.fori_loop(0, (nev + 1) // 2, evac_body, 0)

    return hop_kernel(z, src_r, dst_r)


BN = 2000


def _mm_body(x_ref, w_ref, od_ref, h_ref, z_ref):
    h = jnp.dot(x_ref[...].astype(jnp.bfloat16),
                w_ref[...].astype(jnp.bfloat16),
                preferred_element_type=_f32)
    h_ref[...] = h
    inv_out = 1.0 / jnp.sqrt(jnp.maximum(od_ref[...], 1.0))
    z_ref[...] = h * inv_out


def _tc_matmul(n_feat, weight, out_deg2):
    return pl.pallas_call(
        _mm_body,
        grid=(N // BN,),
        in_specs=[
            pl.BlockSpec((BN, D_IN), lambda i: (i, 0)),
            pl.BlockSpec((D_IN, D_OUT), lambda i: (0, 0)),
            pl.BlockSpec((BN, 1), lambda i: (i, 0)),
        ],
        out_specs=(
            pl.BlockSpec((BN, D_OUT), lambda i: (i, 0)),
            pl.BlockSpec((BN, D_OUT), lambda i: (i, 0)),
        ),
        out_shape=(
            jax.ShapeDtypeStruct((N, D_OUT), _f32),
            jax.ShapeDtypeStruct((N, D_OUT), _f32),
        ),
    )(n_feat, weight, out_deg2)


def _scale_body(p0_ref, p1_ref, id_ref, od_ref, ak_ref, z_ref):
    agg = p0_ref[...] + p1_ref[...]
    inv_in = 1.0 / jnp.sqrt(jnp.maximum(id_ref[...], 1.0))
    inv_out = 1.0 / jnp.sqrt(jnp.maximum(od_ref[...], 1.0))
    ak = agg * inv_in
    ak_ref[...] = ak
    z_ref[...] = ak * inv_out


def _tc_scale(p0, p1, in_deg2, out_deg2):
    return pl.pallas_call(
        _scale_body,
        grid=(N // BN,),
        in_specs=[
            pl.BlockSpec((BN, D_OUT), lambda i: (i, 0)),
            pl.BlockSpec((BN, D_OUT), lambda i: (i, 0)),
            pl.BlockSpec((BN, 1), lambda i: (i, 0)),
            pl.BlockSpec((BN, 1), lambda i: (i, 0)),
        ],
        out_specs=(
            pl.BlockSpec((BN, D_OUT), lambda i: (i, 0)),
            pl.BlockSpec((BN, D_OUT), lambda i: (i, 0)),
        ),
        out_shape=(
            jax.ShapeDtypeStruct((N, D_OUT), _f32),
            jax.ShapeDtypeStruct((N, D_OUT), _f32),
        ),
    )(p0, p1, in_deg2, out_deg2)


def _se_body(a0_ref, a1_ref, a2_ref, p0_ref, p1_ref, id_ref,
             att_ref, w1_ref, w2_ref, out_ref):
    inv_in = 1.0 / jnp.sqrt(jnp.maximum(id_ref[...], 1.0))
    a3 = (p0_ref[...] + p1_ref[...]) * inv_in
    blks = [a0_ref[...], a1_ref[...], a2_ref[...], a3]
    att = att_ref[...].reshape(1, D_OUT)
    s = jnp.concatenate(
        [jnp.sum(b * att, axis=1, keepdims=True) for b in blks], axis=1)
    norm = jnp.sqrt(jnp.sum(s * s, axis=1, keepdims=True))
    s = s / jnp.maximum(norm, 1e-12)
    e = jnp.clip(jnp.dot(s.astype(jnp.bfloat16),
                         w1_ref[...].astype(jnp.bfloat16),
                         preferred_element_type=_f32), 0.0, 6.0)
    e = jnp.tanh(jnp.dot(e.astype(jnp.bfloat16),
                         w2_ref[...].astype(jnp.bfloat16),
                         preferred_element_type=_f32))
    out = blks[0] * e[:, 0:1]
    for k in range(1, K + 1):
        out = out + blks[k] * e[:, k:k + 1]
    out_ref[...] = out


def _tc_se(a0, a1, a2, p0, p1, in_deg2, e_att, e_w1, e_w2):
    return pl.pallas_call(
        _se_body,
        grid=(N // BN,),
        in_specs=[
            pl.BlockSpec((BN, D_OUT), lambda i: (i, 0)),
            pl.BlockSpec((BN, D_OUT), lambda i: (i, 0)),
            pl.BlockSpec((BN, D_OUT), lambda i: (i, 0)),
            pl.BlockSpec((BN, D_OUT), lambda i: (i, 0)),
            pl.BlockSpec((BN, D_OUT), lambda i: (i, 0)),
            pl.BlockSpec((BN, 1), lambda i: (i, 0)),
            pl.BlockSpec((D_OUT, 1), lambda i: (0, 0)),
            pl.BlockSpec((K + 1, K + 1), lambda i: (0, 0)),
            pl.BlockSpec((K + 1, K + 1), lambda i: (0, 0)),
        ],
        out_specs=pl.BlockSpec((BN, D_OUT), lambda i: (i, 0)),
        out_shape=jax.ShapeDtypeStruct((N, D_OUT), _f32),
    )(a0, a1, a2, p0, p1, in_deg2, e_att, e_w1, e_w2)


def kernel(n_feat, edge_index, weight, e_weight1, e_weight2, e_att):
    deg_idx = edge_index.reshape(2, NS, E // NS // DCH, DCH)
    out_deg, in_deg = _sc_degrees(deg_idx)
    od2 = out_deg.reshape(N, 1)
    id2 = in_deg.reshape(N, 1)

    h, z = _tc_matmul(n_feat, weight, od2)

    src_r = edge_index[0].reshape(NW, EPW)
    dst_r = edge_index[1].reshape(NW, NCHUNK, CH)
    aggr = [h]
    for _ in range(K - 1):
        p0, p1 = _sc_hop(z, src_r, dst_r)
        ak, z = _tc_scale(p0, p1, id2, od2)
        aggr.append(ak)
    p0, p1 = _sc_hop(z, src_r, dst_r)

    return _tc_se(aggr[0], aggr[1], aggr[2], p0, p1, id2,
                  e_att, e_weight1, e_weight2)
